# baseline (speedup 1.0000x reference)
import functools

import jax
import jax.numpy as jnp
from jax import lax
from jax.experimental import pallas as pl
from jax.experimental.pallas import tpu as pltpu
from jax.experimental.pallas import tpu_sc as plsc

B = 16
T = 16384
D = 128
NR_PAD = 8
TB = 2048
NBT = T // TB
NA = 5
NB_ = NBT - NA
TA = NA * TB
TBK = NB_ * TB
NEG = -1e30
CLAMP = -1e20


def _sc_info():
    try:
        info = plsc.get_sparse_core_info()
        return info.num_cores, info.num_subcores
    except Exception:
        return 2, 16


def _make_sc_gather(with_qj, tok_base, n_tok):
    NC, NS = _sc_info()
    NW = NC * NS
    rows_per_w = n_tok // NW
    mesh = plsc.VectorSubcoreMesh(core_axis_name="c", subcore_axis_name="s")

    out_type = [jax.ShapeDtypeStruct((n_tok, D), jnp.float32)]
    scratch = [
        pltpu.VMEM((rows_per_w,), jnp.int32),
        pltpu.VMEM((rows_per_w, D), jnp.float32),
        pltpu.SemaphoreType.DMA,
    ]
    if with_qj:
        out_type.append(jax.ShapeDtypeStruct((B, D), jnp.float32))
        scratch += [pltpu.VMEM((B,), jnp.int32), pltpu.VMEM((B, D), jnp.float32)]

    @functools.partial(
        pl.kernel,
        mesh=mesh,
        out_type=out_type,
        scratch_types=scratch,
        compiler_params=pltpu.CompilerParams(use_tc_tiling_on_sc=True),
    )
    def sc_gather(u_table, u_idx, *rest):
        if with_qj:
            i_table, v_idx, pt_out, qj_out, idx_v, rows_v, sem, vidx_v, vrows_v = rest
        else:
            pt_out, idx_v, rows_v, sem = rest
        wid = lax.axis_index("s") * NC + lax.axis_index("c")
        base = wid * rows_per_w
        pltpu.sync_copy(u_idx.at[pl.ds(tok_base + base, rows_per_w)], idx_v)
        pltpu.async_copy(u_table.at[idx_v], rows_v, sem).wait()
        pltpu.sync_copy(rows_v, pt_out.at[pl.ds(base, rows_per_w)])

        if with_qj:
            @pl.when(wid == 0)
            def _():
                pltpu.sync_copy(v_idx, vidx_v)
                pltpu.async_copy(i_table.at[vidx_v], vrows_v, sem).wait()
                pltpu.sync_copy(vrows_v, qj_out)

    return sc_gather


def _dot_t(x, w):
    return lax.dot_general(x, w, (((1,), (1,)), ((), ())),
                           preferred_element_type=jnp.float32)


def _bdot_t(x, w):
    return lax.dot_general(x.astype(jnp.bfloat16), w.astype(jnp.bfloat16),
                           (((1,), (1,)), ((), ())),
                           preferred_element_type=jnp.float32)


def _dot(x, w):
    return lax.dot_general(x, w, (((1,), (0,)), ((), ())),
                           preferred_element_type=jnp.float32)


def _dot00(x, w):
    return lax.dot_general(x, w, (((0,), (0,)), ((), ())),
                           preferred_element_type=jnp.float32)


def _mlp_chunk(pt_ref, seg_ref, rat_ref, g1_ref, g2_ref, a1_ref, a2_ref,
               a3_ref, a3b_ref, g2b_ref, a2b_ref,
               erp_sc, qjp_sc, s_sc, fjt_sc):
    pt = pt_ref[...]
    seg = seg_ref[...].reshape(1, TB)
    rat = rat_ref[...].reshape(1, TB)
    oh_s = (seg == lax.broadcasted_iota(jnp.int32, (B, TB), 0)
            ).astype(jnp.float32)
    oh_r = (rat == lax.broadcasted_iota(jnp.int32, (NR_PAD, TB), 0)
            ).astype(jnp.float32)

    g1 = g1_ref[...]
    a1 = a1_ref[...]

    ec = _dot00(oh_r, erp_sc[...])
    qc = _dot00(oh_s, qjp_sc[...])

    HB = TB // 2
    g2 = g2_ref[...]
    a2 = a2_ref[...]
    a3 = a3_ref[...]
    g2b = g2b_ref[...]
    a2b = a2b_ref[...]
    h1 = jnp.maximum(_bdot_t(pt[:HB], g1[:, :D]) + ec[:HB], 0.0)
    h2 = jnp.maximum(_bdot_t(pt[HB:], g1[:, :D]) + ec[HB:], 0.0)
    f1 = jnp.maximum(_bdot_t(h1, g2) + g2b, 0.0)
    f2 = jnp.maximum(_bdot_t(h2, g2) + g2b, 0.0)
    x1 = jnp.maximum(_bdot_t(f1, a1[:, :D]) + qc[:HB], 0.0)
    x2 = jnp.maximum(_bdot_t(f2, a1[:, :D]) + qc[HB:], 0.0)
    y1 = jnp.maximum(_bdot_t(x1, a2) + a2b, 0.0)
    y2 = jnp.maximum(_bdot_t(x2, a2) + a2b, 0.0)
    s_sc[...] = jnp.concatenate(
        [_bdot_t(a3, y1), _bdot_t(a3, y2)], axis=1) + a3b_ref[0, 0]
    fjt_sc[...] = jnp.concatenate([f1, f2], axis=0)


def _state_update(seg, s_sc, fjt_sc, m_sc, l_sc, zacc_sc):
    oh_s = (seg == lax.broadcasted_iota(jnp.int32, (B, TB), 0)
            ).astype(jnp.float32)
    s_tok = s_sc[...]
    smat = jnp.where(oh_s > 0.0, s_tok, NEG)
    m_old = m_sc[...]
    m_new = jnp.maximum(m_old, jnp.max(smat, axis=1, keepdims=True))
    mc = jnp.maximum(m_new, CLAMP)
    scale = jnp.exp(m_old - mc)
    m_tok = _dot00(mc, oh_s)
    e_tok = jnp.exp(s_tok - m_tok)
    et = oh_s * e_tok
    l_sum = lax.dot_general(oh_s, e_tok, (((1,), (1,)), ((), ())),
                            preferred_element_type=jnp.float32)
    l_sc[...] = l_sc[...] * scale + l_sum
    m_sc[...] = m_new

    eye = (lax.broadcasted_iota(jnp.int32, (B, B), 0)
           == lax.broadcasted_iota(jnp.int32, (B, B), 1))
    dscale = jnp.where(eye, _dot(scale, jnp.ones((1, B), jnp.float32)), 0.0)
    zacc_sc[...] = _dot(dscale, zacc_sc[...]) + _dot(et, fjt_sc[...])


def _proj_init(er_ref, qj_ref, g1_ref, a1_ref, g1b_ref, a1b_ref,
               erp_sc, qjp_sc):
    erp_sc[...] = (_dot_t(er_ref[...], g1_ref[...][:, D:])
                   + g1b_ref[...])
    qjp_sc[...] = (_dot_t(qj_ref[...], a1_ref[...][:, D:])
                   + a1b_ref[...])


def _tc_body_a(pt_ref, qj_ref, seg_ref, rat_ref, segp_ref, er_ref, g1_ref,
               g2_ref, a1_ref, a2_ref, a3_ref, g1b_ref, g2b_ref, a1b_ref,
               a2b_ref, a3b_ref, m_ref, l_ref, zacc_ref, qjp_ref,
               m_sc, l_sc, zacc_sc, erp_sc, qjp_sc, s_sc, fjt_sc):
    i = pl.program_id(0)

    @pl.when(i == 0)
    def _():
        m_sc[...] = jnp.full((B, 1), NEG, jnp.float32)
        l_sc[...] = jnp.zeros((B, 1), jnp.float32)
        zacc_sc[...] = jnp.zeros((B, D), jnp.float32)
        s_sc[...] = jnp.full((1, TB), NEG, jnp.float32)
        fjt_sc[...] = jnp.zeros((TB, D), jnp.float32)
        _proj_init(er_ref, qj_ref, g1_ref, a1_ref, g1b_ref, a1b_ref,
                   erp_sc, qjp_sc)

    _state_update(segp_ref[...].reshape(1, TB), s_sc, fjt_sc,
                  m_sc, l_sc, zacc_sc)

    _mlp_chunk(pt_ref, seg_ref, rat_ref, g1_ref, g2_ref, a1_ref, a2_ref,
               a3_ref, a3b_ref, g2b_ref, a2b_ref,
               erp_sc, qjp_sc, s_sc, fjt_sc)

    @pl.when(i == NA - 1)
    def _():
        _state_update(seg_ref[...].reshape(1, TB), s_sc, fjt_sc,
                      m_sc, l_sc, zacc_sc)
        m_ref[...] = m_sc[...]
        l_ref[...] = l_sc[...]
        zacc_ref[...] = zacc_sc[...]
        qjp_ref[...] = qjp_sc[...]


def _tc_body_b(pt_ref, seg_ref, rat_ref, segp_ref, er_ref, g1_ref, g2_ref,
               a1_ref, a2_ref, a3_ref, g1b_ref, g2b_ref, a1b_ref, a2b_ref,
               a3b_ref, m_in, l_in, zacc_in, qjp_in, z_ref,
               m_sc, l_sc, zacc_sc, erp_sc, qjp_sc, s_sc, fjt_sc):
    i = pl.program_id(0)

    @pl.when(i == 0)
    def _():
        m_sc[...] = m_in[...]
        l_sc[...] = l_in[...]
        zacc_sc[...] = zacc_in[...]
        qjp_sc[...] = qjp_in[...]
        s_sc[...] = jnp.full((1, TB), NEG, jnp.float32)
        fjt_sc[...] = jnp.zeros((TB, D), jnp.float32)
        erp_sc[...] = (_dot_t(er_ref[...], g1_ref[...][:, D:])
                       + g1b_ref[...])

    _state_update(segp_ref[...].reshape(1, TB), s_sc, fjt_sc,
                  m_sc, l_sc, zacc_sc)

    _mlp_chunk(pt_ref, seg_ref, rat_ref, g1_ref, g2_ref, a1_ref, a2_ref,
               a3_ref, a3b_ref, g2b_ref, a2b_ref,
               erp_sc, qjp_sc, s_sc, fjt_sc)

    @pl.when(i == NB_ - 1)
    def _():
        _state_update(seg_ref[...].reshape(1, TB), s_sc, fjt_sc,
                      m_sc, l_sc, zacc_sc)
        eye = (lax.broadcasted_iota(jnp.int32, (B, B), 0)
               == lax.broadcasted_iota(jnp.int32, (B, B), 1))
        recip = 1.0 / jnp.maximum(l_sc[...], 1e-30)
        drec = jnp.where(eye, _dot(recip, jnp.ones((1, B), jnp.float32)), 0.0)
        z_ref[...] = _dot(drec, zacc_sc[...])


_SCRATCH = [
    pltpu.VMEM((B, 1), jnp.float32),
    pltpu.VMEM((B, 1), jnp.float32),
    pltpu.VMEM((B, D), jnp.float32),
    pltpu.VMEM((NR_PAD, D), jnp.float32),
    pltpu.VMEM((B, D), jnp.float32),
    pltpu.VMEM((1, TB), jnp.float32),
    pltpu.VMEM((TB, D), jnp.float32),
]

_full = lambda s: pl.BlockSpec(s, lambda i: tuple(0 for _ in s))
_W_SPECS = [
    _full((NR_PAD, D)), _full((D, 2 * D)), _full((D, D)), _full((D, 2 * D)),
    _full((D, D)), _full((1, D)), _full((1, D)), _full((1, D)), _full((1, D)),
    _full((1, D)), _full((1, 1)),
]
_STATE_SHAPES = [
    jax.ShapeDtypeStruct((B, 1), jnp.float32),
    jax.ShapeDtypeStruct((B, 1), jnp.float32),
    jax.ShapeDtypeStruct((B, D), jnp.float32),
    jax.ShapeDtypeStruct((B, D), jnp.float32),
]
_STATE_SPECS = [_full((B, 1)), _full((B, 1)), _full((B, D)), _full((B, D))]


def _tc_pass_a(pt, qj, seg3, rat3, er_pad, *weights):
    return pl.pallas_call(
        _tc_body_a,
        grid=(NA,),
        in_specs=[
            pl.BlockSpec((TB, D), lambda i: (i, 0)),
            _full((B, D)),
            pl.BlockSpec((1, 1, TB), lambda i: (i, 0, 0)),
            pl.BlockSpec((1, 1, TB), lambda i: (i, 0, 0)),
            pl.BlockSpec((1, 1, TB), lambda i: (jnp.maximum(i - 1, 0), 0, 0)),
        ] + _W_SPECS,
        out_specs=_STATE_SPECS,
        out_shape=_STATE_SHAPES,
        scratch_shapes=_SCRATCH,
    )(pt, qj, seg3, rat3, seg3, er_pad, *weights)


def _tc_pass_b(pt, seg3, rat3, er_pad, state, *weights):
    return pl.pallas_call(
        _tc_body_b,
        grid=(NB_,),
        in_specs=[
            pl.BlockSpec((TB, D), lambda i: (i, 0)),
            pl.BlockSpec((1, 1, TB), lambda i: (i + NA, 0, 0)),
            pl.BlockSpec((1, 1, TB), lambda i: (i + NA, 0, 0)),
            pl.BlockSpec((1, 1, TB),
                         lambda i: (jnp.maximum(i - 1, 0) + NA, 0, 0)),
        ] + _W_SPECS + _STATE_SPECS,
        out_specs=_full((B, D)),
        out_shape=jax.ShapeDtypeStruct((B, D), jnp.float32),
        scratch_shapes=_SCRATCH,
    )(pt, seg3, rat3, seg3, er_pad, *weights, *state)


def kernel(nodes_v, flat_users, flat_ratings, segment_ids, embed_u_w,
           embed_i_w, embed_r_w, g1_w, g1_b, g2_w, g2_b, a1_w, a1_b,
           a2_w, a2_b, a3_w, a3_b):
    sc_a = _make_sc_gather(True, 0, TA)
    sc_b = _make_sc_gather(False, TA, TBK)
    pt_a, qj = sc_a(embed_u_w, flat_users, embed_i_w, nodes_v)
    pt_b, = sc_b(embed_u_w, flat_users)

    seg3 = segment_ids.reshape(NBT, 1, TB)
    rat3 = flat_ratings.reshape(NBT, 1, TB)
    er_pad = jnp.zeros((NR_PAD, D), jnp.float32).at[:5].set(embed_r_w)
    weights = (g1_w, g2_w, a1_w, a2_w, a3_w,
               g1_b.reshape(1, D), g2_b.reshape(1, D), a1_b.reshape(1, D),
               a2_b.reshape(1, D), a3_b.reshape(1, 1))

    state = _tc_pass_a(pt_a, qj, seg3, rat3, er_pad, *weights)
    return _tc_pass_b(pt_b, seg3, rat3, er_pad, state, *weights)


# speedup vs baseline: 1.0176x; 1.0176x over previous
import functools

import jax
import jax.numpy as jnp
from jax import lax
from jax.experimental import pallas as pl
from jax.experimental.pallas import tpu as pltpu
from jax.experimental.pallas import tpu_sc as plsc

B = 16
T = 16384
D = 128
NR_PAD = 8
TB = 2048
NBT = T // TB
NA = 4
NB_ = NBT - NA
TA = NA * TB
TBK = NB_ * TB
NEG = -1e30
CLAMP = -1e20


def _sc_info():
    try:
        info = plsc.get_sparse_core_info()
        return info.num_cores, info.num_subcores
    except Exception:
        return 2, 16


def _make_sc_gather(with_qj, tok_base, n_tok):
    NC, NS = _sc_info()
    NW = NC * NS
    rows_per_w = n_tok // NW
    mesh = plsc.VectorSubcoreMesh(core_axis_name="c", subcore_axis_name="s")

    out_type = [jax.ShapeDtypeStruct((n_tok, D), jnp.float32)]
    scratch = [
        pltpu.VMEM((rows_per_w,), jnp.int32),
        pltpu.VMEM((rows_per_w, D), jnp.float32),
        pltpu.SemaphoreType.DMA,
    ]
    if with_qj:
        out_type.append(jax.ShapeDtypeStruct((B, D), jnp.float32))
        scratch += [pltpu.VMEM((B,), jnp.int32), pltpu.VMEM((B, D), jnp.float32)]

    @functools.partial(
        pl.kernel,
        mesh=mesh,
        out_type=out_type,
        scratch_types=scratch,
        compiler_params=pltpu.CompilerParams(use_tc_tiling_on_sc=True),
    )
    def sc_gather(u_table, u_idx, *rest):
        if with_qj:
            i_table, v_idx, pt_out, qj_out, idx_v, rows_v, sem, vidx_v, vrows_v = rest
        else:
            pt_out, idx_v, rows_v, sem = rest
        wid = lax.axis_index("s") * NC + lax.axis_index("c")
        base = wid * rows_per_w
        pltpu.sync_copy(u_idx.at[pl.ds(tok_base + base, rows_per_w)], idx_v)
        pltpu.async_copy(u_table.at[idx_v], rows_v, sem).wait()
        pltpu.sync_copy(rows_v, pt_out.at[pl.ds(base, rows_per_w)])

        if with_qj:
            @pl.when(wid == 0)
            def _():
                pltpu.sync_copy(v_idx, vidx_v)
                pltpu.async_copy(i_table.at[vidx_v], vrows_v, sem).wait()
                pltpu.sync_copy(vrows_v, qj_out)

    return sc_gather


def _dot_t(x, w):
    return lax.dot_general(x, w, (((1,), (1,)), ((), ())),
                           preferred_element_type=jnp.float32)


def _bdot_t(x, w):
    return lax.dot_general(x.astype(jnp.bfloat16), w.astype(jnp.bfloat16),
                           (((1,), (1,)), ((), ())),
                           preferred_element_type=jnp.float32)


def _dot(x, w):
    return lax.dot_general(x, w, (((1,), (0,)), ((), ())),
                           preferred_element_type=jnp.float32)


def _dot00(x, w):
    return lax.dot_general(x, w, (((0,), (0,)), ((), ())),
                           preferred_element_type=jnp.float32)


def _mlp_chunk(pt_ref, seg_ref, rat_ref, g1_ref, g2_ref, a1_ref, a2_ref,
               a3_ref, a3b_ref, g2b_ref, a2b_ref,
               erp_sc, qjp_sc, s_sc, fjt_sc):
    pt = pt_ref[...]
    seg = seg_ref[...].reshape(1, TB)
    rat = rat_ref[...].reshape(1, TB)
    oh_s = (seg == lax.broadcasted_iota(jnp.int32, (B, TB), 0)
            ).astype(jnp.float32)
    oh_r = (rat == lax.broadcasted_iota(jnp.int32, (NR_PAD, TB), 0)
            ).astype(jnp.float32)

    g1 = g1_ref[...]
    a1 = a1_ref[...]

    ec = _dot00(oh_r, erp_sc[...])
    qc = _dot00(oh_s, qjp_sc[...])

    HB = TB // 2
    g2 = g2_ref[...]
    a2 = a2_ref[...]
    a3 = a3_ref[...]
    g2b = g2b_ref[...]
    a2b = a2b_ref[...]
    h1 = jnp.maximum(_bdot_t(pt[:HB], g1[:, :D]) + ec[:HB], 0.0)
    h2 = jnp.maximum(_bdot_t(pt[HB:], g1[:, :D]) + ec[HB:], 0.0)
    f1 = jnp.maximum(_bdot_t(h1, g2) + g2b, 0.0)
    f2 = jnp.maximum(_bdot_t(h2, g2) + g2b, 0.0)
    x1 = jnp.maximum(_bdot_t(f1, a1[:, :D]) + qc[:HB], 0.0)
    x2 = jnp.maximum(_bdot_t(f2, a1[:, :D]) + qc[HB:], 0.0)
    y1 = jnp.maximum(_bdot_t(x1, a2) + a2b, 0.0)
    y2 = jnp.maximum(_bdot_t(x2, a2) + a2b, 0.0)
    s_sc[...] = jnp.concatenate(
        [_bdot_t(a3, y1), _bdot_t(a3, y2)], axis=1) + a3b_ref[0, 0]
    fjt_sc[...] = jnp.concatenate([f1, f2], axis=0)


def _state_update(seg, s_sc, fjt_sc, m_sc, l_sc, zacc_sc):
    oh_s = (seg == lax.broadcasted_iota(jnp.int32, (B, TB), 0)
            ).astype(jnp.float32)
    s_tok = s_sc[...]
    smat = jnp.where(oh_s > 0.0, s_tok, NEG)
    m_old = m_sc[...]
    m_new = jnp.maximum(m_old, jnp.max(smat, axis=1, keepdims=True))
    mc = jnp.maximum(m_new, CLAMP)
    scale = jnp.exp(m_old - mc)
    m_tok = _dot00(mc, oh_s)
    e_tok = jnp.exp(s_tok - m_tok)
    et = oh_s * e_tok
    l_sum = lax.dot_general(oh_s, e_tok, (((1,), (1,)), ((), ())),
                            preferred_element_type=jnp.float32)
    l_sc[...] = l_sc[...] * scale + l_sum
    m_sc[...] = m_new

    eye = (lax.broadcasted_iota(jnp.int32, (B, B), 0)
           == lax.broadcasted_iota(jnp.int32, (B, B), 1))
    dscale = jnp.where(eye, _dot(scale, jnp.ones((1, B), jnp.float32)), 0.0)
    zacc_sc[...] = _dot(dscale, zacc_sc[...]) + _dot(et, fjt_sc[...])


def _proj_init(er_ref, qj_ref, g1_ref, a1_ref, g1b_ref, a1b_ref,
               erp_sc, qjp_sc):
    erp_sc[...] = (_dot_t(er_ref[...], g1_ref[...][:, D:])
                   + g1b_ref[...])
    qjp_sc[...] = (_dot_t(qj_ref[...], a1_ref[...][:, D:])
                   + a1b_ref[...])


def _tc_body_a(pt_ref, qj_ref, seg_ref, rat_ref, segp_ref, er_ref, g1_ref,
               g2_ref, a1_ref, a2_ref, a3_ref, g1b_ref, g2b_ref, a1b_ref,
               a2b_ref, a3b_ref, m_ref, l_ref, zacc_ref, qjp_ref,
               m_sc, l_sc, zacc_sc, erp_sc, qjp_sc, s_sc, fjt_sc):
    i = pl.program_id(0)

    @pl.when(i == 0)
    def _():
        m_sc[...] = jnp.full((B, 1), NEG, jnp.float32)
        l_sc[...] = jnp.zeros((B, 1), jnp.float32)
        zacc_sc[...] = jnp.zeros((B, D), jnp.float32)
        s_sc[...] = jnp.full((1, TB), NEG, jnp.float32)
        fjt_sc[...] = jnp.zeros((TB, D), jnp.float32)
        _proj_init(er_ref, qj_ref, g1_ref, a1_ref, g1b_ref, a1b_ref,
                   erp_sc, qjp_sc)

    _state_update(segp_ref[...].reshape(1, TB), s_sc, fjt_sc,
                  m_sc, l_sc, zacc_sc)

    _mlp_chunk(pt_ref, seg_ref, rat_ref, g1_ref, g2_ref, a1_ref, a2_ref,
               a3_ref, a3b_ref, g2b_ref, a2b_ref,
               erp_sc, qjp_sc, s_sc, fjt_sc)

    @pl.when(i == NA - 1)
    def _():
        _state_update(seg_ref[...].reshape(1, TB), s_sc, fjt_sc,
                      m_sc, l_sc, zacc_sc)
        m_ref[...] = m_sc[...]
        l_ref[...] = l_sc[...]
        zacc_ref[...] = zacc_sc[...]
        qjp_ref[...] = qjp_sc[...]


def _tc_body_b(pt_ref, seg_ref, rat_ref, segp_ref, er_ref, g1_ref, g2_ref,
               a1_ref, a2_ref, a3_ref, g1b_ref, g2b_ref, a1b_ref, a2b_ref,
               a3b_ref, m_in, l_in, zacc_in, qjp_in, z_ref,
               m_sc, l_sc, zacc_sc, erp_sc, qjp_sc, s_sc, fjt_sc):
    i = pl.program_id(0)

    @pl.when(i == 0)
    def _():
        m_sc[...] = m_in[...]
        l_sc[...] = l_in[...]
        zacc_sc[...] = zacc_in[...]
        qjp_sc[...] = qjp_in[...]
        s_sc[...] = jnp.full((1, TB), NEG, jnp.float32)
        fjt_sc[...] = jnp.zeros((TB, D), jnp.float32)
        erp_sc[...] = (_dot_t(er_ref[...], g1_ref[...][:, D:])
                       + g1b_ref[...])

    _state_update(segp_ref[...].reshape(1, TB), s_sc, fjt_sc,
                  m_sc, l_sc, zacc_sc)

    _mlp_chunk(pt_ref, seg_ref, rat_ref, g1_ref, g2_ref, a1_ref, a2_ref,
               a3_ref, a3b_ref, g2b_ref, a2b_ref,
               erp_sc, qjp_sc, s_sc, fjt_sc)

    @pl.when(i == NB_ - 1)
    def _():
        _state_update(seg_ref[...].reshape(1, TB), s_sc, fjt_sc,
                      m_sc, l_sc, zacc_sc)
        eye = (lax.broadcasted_iota(jnp.int32, (B, B), 0)
               == lax.broadcasted_iota(jnp.int32, (B, B), 1))
        recip = 1.0 / jnp.maximum(l_sc[...], 1e-30)
        drec = jnp.where(eye, _dot(recip, jnp.ones((1, B), jnp.float32)), 0.0)
        z_ref[...] = _dot(drec, zacc_sc[...])


_SCRATCH = [
    pltpu.VMEM((B, 1), jnp.float32),
    pltpu.VMEM((B, 1), jnp.float32),
    pltpu.VMEM((B, D), jnp.float32),
    pltpu.VMEM((NR_PAD, D), jnp.float32),
    pltpu.VMEM((B, D), jnp.float32),
    pltpu.VMEM((1, TB), jnp.float32),
    pltpu.VMEM((TB, D), jnp.float32),
]

_full = lambda s: pl.BlockSpec(s, lambda i: tuple(0 for _ in s))
_W_SPECS = [
    _full((NR_PAD, D)), _full((D, 2 * D)), _full((D, D)), _full((D, 2 * D)),
    _full((D, D)), _full((1, D)), _full((1, D)), _full((1, D)), _full((1, D)),
    _full((1, D)), _full((1, 1)),
]
_STATE_SHAPES = [
    jax.ShapeDtypeStruct((B, 1), jnp.float32),
    jax.ShapeDtypeStruct((B, 1), jnp.float32),
    jax.ShapeDtypeStruct((B, D), jnp.float32),
    jax.ShapeDtypeStruct((B, D), jnp.float32),
]
_STATE_SPECS = [_full((B, 1)), _full((B, 1)), _full((B, D)), _full((B, D))]


def _tc_pass_a(pt, qj, seg3, rat3, er_pad, *weights):
    return pl.pallas_call(
        _tc_body_a,
        grid=(NA,),
        in_specs=[
            pl.BlockSpec((TB, D), lambda i: (i, 0)),
            _full((B, D)),
            pl.BlockSpec((1, 1, TB), lambda i: (i, 0, 0)),
            pl.BlockSpec((1, 1, TB), lambda i: (i, 0, 0)),
            pl.BlockSpec((1, 1, TB), lambda i: (jnp.maximum(i - 1, 0), 0, 0)),
        ] + _W_SPECS,
        out_specs=_STATE_SPECS,
        out_shape=_STATE_SHAPES,
        scratch_shapes=_SCRATCH,
    )(pt, qj, seg3, rat3, seg3, er_pad, *weights)


def _tc_pass_b(pt, seg3, rat3, er_pad, state, *weights):
    return pl.pallas_call(
        _tc_body_b,
        grid=(NB_,),
        in_specs=[
            pl.BlockSpec((TB, D), lambda i: (i, 0)),
            pl.BlockSpec((1, 1, TB), lambda i: (i + NA, 0, 0)),
            pl.BlockSpec((1, 1, TB), lambda i: (i + NA, 0, 0)),
            pl.BlockSpec((1, 1, TB),
                         lambda i: (jnp.maximum(i - 1, 0) + NA, 0, 0)),
        ] + _W_SPECS + _STATE_SPECS,
        out_specs=_full((B, D)),
        out_shape=jax.ShapeDtypeStruct((B, D), jnp.float32),
        scratch_shapes=_SCRATCH,
    )(pt, seg3, rat3, seg3, er_pad, *weights, *state)


def kernel(nodes_v, flat_users, flat_ratings, segment_ids, embed_u_w,
           embed_i_w, embed_r_w, g1_w, g1_b, g2_w, g2_b, a1_w, a1_b,
           a2_w, a2_b, a3_w, a3_b):
    sc_a = _make_sc_gather(True, 0, TA)
    sc_b = _make_sc_gather(False, TA, TBK)
    pt_a, qj = sc_a(embed_u_w, flat_users, embed_i_w, nodes_v)
    pt_b, = sc_b(embed_u_w, flat_users)

    seg3 = segment_ids.reshape(NBT, 1, TB)
    rat3 = flat_ratings.reshape(NBT, 1, TB)
    er_pad = jnp.zeros((NR_PAD, D), jnp.float32).at[:5].set(embed_r_w)
    weights = (g1_w, g2_w, a1_w, a2_w, a3_w,
               g1_b.reshape(1, D), g2_b.reshape(1, D), a1_b.reshape(1, D),
               a2_b.reshape(1, D), a3_b.reshape(1, 1))

    state = _tc_pass_a(pt_a, qj, seg3, rat3, er_pad, *weights)
    return _tc_pass_b(pt_b, seg3, rat3, er_pad, state, *weights)


# speedup vs baseline: 1.0554x; 1.0372x over previous
import functools

import jax
import jax.numpy as jnp
from jax import lax
from jax.experimental import pallas as pl
from jax.experimental.pallas import tpu as pltpu
from jax.experimental.pallas import tpu_sc as plsc

B = 16
T = 16384
D = 128
NR_PAD = 8
TB = 4096
NBT = T // TB
NA = 2
NB_ = NBT - NA
TA = NA * TB
TBK = NB_ * TB
NEG = -1e30
CLAMP = -1e20


def _sc_info():
    try:
        info = plsc.get_sparse_core_info()
        return info.num_cores, info.num_subcores
    except Exception:
        return 2, 16


def _make_sc_gather(with_qj, tok_base, n_tok):
    NC, NS = _sc_info()
    NW = NC * NS
    rows_per_w = n_tok // NW
    mesh = plsc.VectorSubcoreMesh(core_axis_name="c", subcore_axis_name="s")

    out_type = [jax.ShapeDtypeStruct((n_tok, D), jnp.float32)]
    scratch = [
        pltpu.VMEM((rows_per_w,), jnp.int32),
        pltpu.VMEM((rows_per_w, D), jnp.float32),
        pltpu.SemaphoreType.DMA,
    ]
    if with_qj:
        out_type.append(jax.ShapeDtypeStruct((B, D), jnp.float32))
        scratch += [pltpu.VMEM((B,), jnp.int32), pltpu.VMEM((B, D), jnp.float32)]

    @functools.partial(
        pl.kernel,
        mesh=mesh,
        out_type=out_type,
        scratch_types=scratch,
        compiler_params=pltpu.CompilerParams(use_tc_tiling_on_sc=True),
    )
    def sc_gather(u_table, u_idx, *rest):
        if with_qj:
            i_table, v_idx, pt_out, qj_out, idx_v, rows_v, sem, vidx_v, vrows_v = rest
        else:
            pt_out, idx_v, rows_v, sem = rest
        wid = lax.axis_index("s") * NC + lax.axis_index("c")
        base = wid * rows_per_w
        pltpu.sync_copy(u_idx.at[pl.ds(tok_base + base, rows_per_w)], idx_v)
        pltpu.async_copy(u_table.at[idx_v], rows_v, sem).wait()
        pltpu.sync_copy(rows_v, pt_out.at[pl.ds(base, rows_per_w)])

        if with_qj:
            @pl.when(wid == 0)
            def _():
                pltpu.sync_copy(v_idx, vidx_v)
                pltpu.async_copy(i_table.at[vidx_v], vrows_v, sem).wait()
                pltpu.sync_copy(vrows_v, qj_out)

    return sc_gather


def _dot_t(x, w):
    return lax.dot_general(x, w, (((1,), (1,)), ((), ())),
                           preferred_element_type=jnp.float32)


def _bdot_t(x, w):
    return lax.dot_general(x.astype(jnp.bfloat16), w.astype(jnp.bfloat16),
                           (((1,), (1,)), ((), ())),
                           preferred_element_type=jnp.float32)


def _dot(x, w):
    return lax.dot_general(x, w, (((1,), (0,)), ((), ())),
                           preferred_element_type=jnp.float32)


def _dot00(x, w):
    return lax.dot_general(x, w, (((0,), (0,)), ((), ())),
                           preferred_element_type=jnp.float32)


def _mlp_chunk(pt_ref, seg_ref, rat_ref, g1_ref, g2_ref, a1_ref, a2_ref,
               a3_ref, a3b_ref, g2b_ref, a2b_ref,
               erp_sc, qjp_sc, s_sc, fjt_sc):
    pt = pt_ref[...]
    seg = seg_ref[...].reshape(1, TB)
    rat = rat_ref[...].reshape(1, TB)
    oh_s = (seg == lax.broadcasted_iota(jnp.int32, (B, TB), 0)
            ).astype(jnp.float32)
    oh_r = (rat == lax.broadcasted_iota(jnp.int32, (NR_PAD, TB), 0)
            ).astype(jnp.float32)

    g1 = g1_ref[...]
    a1 = a1_ref[...]

    ec = _dot00(oh_r, erp_sc[...])
    qc = _dot00(oh_s, qjp_sc[...])

    HB = TB // 2
    g2 = g2_ref[...]
    a2 = a2_ref[...]
    a3 = a3_ref[...]
    g2b = g2b_ref[...]
    a2b = a2b_ref[...]
    h1 = jnp.maximum(_bdot_t(pt[:HB], g1[:, :D]) + ec[:HB], 0.0)
    h2 = jnp.maximum(_bdot_t(pt[HB:], g1[:, :D]) + ec[HB:], 0.0)
    f1 = jnp.maximum(_bdot_t(h1, g2) + g2b, 0.0)
    f2 = jnp.maximum(_bdot_t(h2, g2) + g2b, 0.0)
    x1 = jnp.maximum(_bdot_t(f1, a1[:, :D]) + qc[:HB], 0.0)
    x2 = jnp.maximum(_bdot_t(f2, a1[:, :D]) + qc[HB:], 0.0)
    y1 = jnp.maximum(_bdot_t(x1, a2) + a2b, 0.0)
    y2 = jnp.maximum(_bdot_t(x2, a2) + a2b, 0.0)
    s_sc[...] = jnp.concatenate(
        [_bdot_t(a3, y1), _bdot_t(a3, y2)], axis=1) + a3b_ref[0, 0]
    fjt_sc[...] = jnp.concatenate([f1, f2], axis=0)


def _state_update(seg, s_sc, fjt_sc, m_sc, l_sc, zacc_sc):
    oh_s = (seg == lax.broadcasted_iota(jnp.int32, (B, TB), 0)
            ).astype(jnp.float32)
    s_tok = s_sc[...]
    smat = jnp.where(oh_s > 0.0, s_tok, NEG)
    m_old = m_sc[...]
    m_new = jnp.maximum(m_old, jnp.max(smat, axis=1, keepdims=True))
    mc = jnp.maximum(m_new, CLAMP)
    scale = jnp.exp(m_old - mc)
    m_tok = _dot00(mc, oh_s)
    e_tok = jnp.exp(s_tok - m_tok)
    et = oh_s * e_tok
    l_sum = lax.dot_general(oh_s, e_tok, (((1,), (1,)), ((), ())),
                            preferred_element_type=jnp.float32)
    l_sc[...] = l_sc[...] * scale + l_sum
    m_sc[...] = m_new

    eye = (lax.broadcasted_iota(jnp.int32, (B, B), 0)
           == lax.broadcasted_iota(jnp.int32, (B, B), 1))
    dscale = jnp.where(eye, _dot(scale, jnp.ones((1, B), jnp.float32)), 0.0)
    zacc_sc[...] = _dot(dscale, zacc_sc[...]) + _dot(et, fjt_sc[...])


def _proj_init(er_ref, qj_ref, g1_ref, a1_ref, g1b_ref, a1b_ref,
               erp_sc, qjp_sc):
    erp_sc[...] = (_dot_t(er_ref[...], g1_ref[...][:, D:])
                   + g1b_ref[...])
    qjp_sc[...] = (_dot_t(qj_ref[...], a1_ref[...][:, D:])
                   + a1b_ref[...])


def _tc_body_a(pt_ref, qj_ref, seg_ref, rat_ref, segp_ref, er_ref, g1_ref,
               g2_ref, a1_ref, a2_ref, a3_ref, g1b_ref, g2b_ref, a1b_ref,
               a2b_ref, a3b_ref, m_ref, l_ref, zacc_ref, qjp_ref,
               m_sc, l_sc, zacc_sc, erp_sc, qjp_sc, s_sc, fjt_sc):
    i = pl.program_id(0)

    @pl.when(i == 0)
    def _():
        m_sc[...] = jnp.full((B, 1), NEG, jnp.float32)
        l_sc[...] = jnp.zeros((B, 1), jnp.float32)
        zacc_sc[...] = jnp.zeros((B, D), jnp.float32)
        s_sc[...] = jnp.full((1, TB), NEG, jnp.float32)
        fjt_sc[...] = jnp.zeros((TB, D), jnp.float32)
        _proj_init(er_ref, qj_ref, g1_ref, a1_ref, g1b_ref, a1b_ref,
                   erp_sc, qjp_sc)

    _state_update(segp_ref[...].reshape(1, TB), s_sc, fjt_sc,
                  m_sc, l_sc, zacc_sc)

    _mlp_chunk(pt_ref, seg_ref, rat_ref, g1_ref, g2_ref, a1_ref, a2_ref,
               a3_ref, a3b_ref, g2b_ref, a2b_ref,
               erp_sc, qjp_sc, s_sc, fjt_sc)

    @pl.when(i == NA - 1)
    def _():
        _state_update(seg_ref[...].reshape(1, TB), s_sc, fjt_sc,
                      m_sc, l_sc, zacc_sc)
        m_ref[...] = m_sc[...]
        l_ref[...] = l_sc[...]
        zacc_ref[...] = zacc_sc[...]
        qjp_ref[...] = qjp_sc[...]


def _tc_body_b(pt_ref, seg_ref, rat_ref, segp_ref, er_ref, g1_ref, g2_ref,
               a1_ref, a2_ref, a3_ref, g1b_ref, g2b_ref, a1b_ref, a2b_ref,
               a3b_ref, m_in, l_in, zacc_in, qjp_in, z_ref,
               m_sc, l_sc, zacc_sc, erp_sc, qjp_sc, s_sc, fjt_sc):
    i = pl.program_id(0)

    @pl.when(i == 0)
    def _():
        m_sc[...] = m_in[...]
        l_sc[...] = l_in[...]
        zacc_sc[...] = zacc_in[...]
        qjp_sc[...] = qjp_in[...]
        s_sc[...] = jnp.full((1, TB), NEG, jnp.float32)
        fjt_sc[...] = jnp.zeros((TB, D), jnp.float32)
        erp_sc[...] = (_dot_t(er_ref[...], g1_ref[...][:, D:])
                       + g1b_ref[...])

    _state_update(segp_ref[...].reshape(1, TB), s_sc, fjt_sc,
                  m_sc, l_sc, zacc_sc)

    _mlp_chunk(pt_ref, seg_ref, rat_ref, g1_ref, g2_ref, a1_ref, a2_ref,
               a3_ref, a3b_ref, g2b_ref, a2b_ref,
               erp_sc, qjp_sc, s_sc, fjt_sc)

    @pl.when(i == NB_ - 1)
    def _():
        _state_update(seg_ref[...].reshape(1, TB), s_sc, fjt_sc,
                      m_sc, l_sc, zacc_sc)
        eye = (lax.broadcasted_iota(jnp.int32, (B, B), 0)
               == lax.broadcasted_iota(jnp.int32, (B, B), 1))
        recip = 1.0 / jnp.maximum(l_sc[...], 1e-30)
        drec = jnp.where(eye, _dot(recip, jnp.ones((1, B), jnp.float32)), 0.0)
        z_ref[...] = _dot(drec, zacc_sc[...])


_SCRATCH = [
    pltpu.VMEM((B, 1), jnp.float32),
    pltpu.VMEM((B, 1), jnp.float32),
    pltpu.VMEM((B, D), jnp.float32),
    pltpu.VMEM((NR_PAD, D), jnp.float32),
    pltpu.VMEM((B, D), jnp.float32),
    pltpu.VMEM((1, TB), jnp.float32),
    pltpu.VMEM((TB, D), jnp.float32),
]

_full = lambda s: pl.BlockSpec(s, lambda i: tuple(0 for _ in s))
_W_SPECS = [
    _full((NR_PAD, D)), _full((D, 2 * D)), _full((D, D)), _full((D, 2 * D)),
    _full((D, D)), _full((1, D)), _full((1, D)), _full((1, D)), _full((1, D)),
    _full((1, D)), _full((1, 1)),
]
_STATE_SHAPES = [
    jax.ShapeDtypeStruct((B, 1), jnp.float32),
    jax.ShapeDtypeStruct((B, 1), jnp.float32),
    jax.ShapeDtypeStruct((B, D), jnp.float32),
    jax.ShapeDtypeStruct((B, D), jnp.float32),
]
_STATE_SPECS = [_full((B, 1)), _full((B, 1)), _full((B, D)), _full((B, D))]


def _tc_pass_a(pt, qj, seg3, rat3, er_pad, *weights):
    return pl.pallas_call(
        _tc_body_a,
        grid=(NA,),
        in_specs=[
            pl.BlockSpec((TB, D), lambda i: (i, 0)),
            _full((B, D)),
            pl.BlockSpec((1, 1, TB), lambda i: (i, 0, 0)),
            pl.BlockSpec((1, 1, TB), lambda i: (i, 0, 0)),
            pl.BlockSpec((1, 1, TB), lambda i: (jnp.maximum(i - 1, 0), 0, 0)),
        ] + _W_SPECS,
        out_specs=_STATE_SPECS,
        out_shape=_STATE_SHAPES,
        scratch_shapes=_SCRATCH,
    )(pt, qj, seg3, rat3, seg3, er_pad, *weights)


def _tc_pass_b(pt, seg3, rat3, er_pad, state, *weights):
    return pl.pallas_call(
        _tc_body_b,
        grid=(NB_,),
        in_specs=[
            pl.BlockSpec((TB, D), lambda i: (i, 0)),
            pl.BlockSpec((1, 1, TB), lambda i: (i + NA, 0, 0)),
            pl.BlockSpec((1, 1, TB), lambda i: (i + NA, 0, 0)),
            pl.BlockSpec((1, 1, TB),
                         lambda i: (jnp.maximum(i - 1, 0) + NA, 0, 0)),
        ] + _W_SPECS + _STATE_SPECS,
        out_specs=_full((B, D)),
        out_shape=jax.ShapeDtypeStruct((B, D), jnp.float32),
        scratch_shapes=_SCRATCH,
    )(pt, seg3, rat3, seg3, er_pad, *weights, *state)


def kernel(nodes_v, flat_users, flat_ratings, segment_ids, embed_u_w,
           embed_i_w, embed_r_w, g1_w, g1_b, g2_w, g2_b, a1_w, a1_b,
           a2_w, a2_b, a3_w, a3_b):
    sc_a = _make_sc_gather(True, 0, TA)
    sc_b = _make_sc_gather(False, TA, TBK)
    pt_a, qj = sc_a(embed_u_w, flat_users, embed_i_w, nodes_v)
    pt_b, = sc_b(embed_u_w, flat_users)

    seg3 = segment_ids.reshape(NBT, 1, TB)
    rat3 = flat_ratings.reshape(NBT, 1, TB)
    er_pad = jnp.zeros((NR_PAD, D), jnp.float32).at[:5].set(embed_r_w)
    weights = (g1_w, g2_w, a1_w, a2_w, a3_w,
               g1_b.reshape(1, D), g2_b.reshape(1, D), a1_b.reshape(1, D),
               a2_b.reshape(1, D), a3_b.reshape(1, 1))

    state = _tc_pass_a(pt_a, qj, seg3, rat3, er_pad, *weights)
    return _tc_pass_b(pt_b, seg3, rat3, er_pad, state, *weights)


# speedup vs baseline: 1.0615x; 1.0057x over previous
import functools

import jax
import jax.numpy as jnp
from jax import lax
from jax.experimental import pallas as pl
from jax.experimental.pallas import tpu as pltpu
from jax.experimental.pallas import tpu_sc as plsc

B = 16
T = 16384
D = 128
NR_PAD = 8
TB = 4096
NBT = T // TB
NA = 2
NB_ = NBT - NA
TA = NA * TB
TBK = NB_ * TB
NEG = -1e30
CLAMP = -1e20


def _sc_info():
    try:
        info = plsc.get_sparse_core_info()
        return info.num_cores, info.num_subcores
    except Exception:
        return 2, 16


def _make_sc_gather(with_qj, tok_base, n_tok):
    NC, NS = _sc_info()
    NW = NC * NS
    rows_per_w = n_tok // NW
    mesh = plsc.VectorSubcoreMesh(core_axis_name="c", subcore_axis_name="s")

    out_type = [jax.ShapeDtypeStruct((n_tok, D), jnp.float32)]
    scratch = [
        pltpu.VMEM((rows_per_w,), jnp.int32),
        pltpu.VMEM((rows_per_w, D), jnp.float32),
        pltpu.SemaphoreType.DMA,
    ]
    if with_qj:
        out_type.append(jax.ShapeDtypeStruct((B, D), jnp.float32))
        scratch += [pltpu.VMEM((B,), jnp.int32), pltpu.VMEM((B, D), jnp.float32)]

    @functools.partial(
        pl.kernel,
        mesh=mesh,
        out_type=out_type,
        scratch_types=scratch,
        compiler_params=pltpu.CompilerParams(use_tc_tiling_on_sc=True),
    )
    def sc_gather(u_table, u_idx, *rest):
        if with_qj:
            i_table, v_idx, pt_out, qj_out, idx_v, rows_v, sem, vidx_v, vrows_v = rest
        else:
            pt_out, idx_v, rows_v, sem = rest
        wid = lax.axis_index("s") * NC + lax.axis_index("c")
        base = wid * rows_per_w
        pltpu.sync_copy(u_idx.at[pl.ds(tok_base + base, rows_per_w)], idx_v)
        pltpu.async_copy(u_table.at[idx_v], rows_v, sem).wait()
        pltpu.sync_copy(rows_v, pt_out.at[pl.ds(base, rows_per_w)])

        if with_qj:
            @pl.when(wid == 0)
            def _():
                pltpu.sync_copy(v_idx, vidx_v)
                pltpu.async_copy(i_table.at[vidx_v], vrows_v, sem).wait()
                pltpu.sync_copy(vrows_v, qj_out)

    return sc_gather


def _dot_t(x, w):
    return lax.dot_general(x, w, (((1,), (1,)), ((), ())),
                           preferred_element_type=jnp.float32)


def _bdot_t(x, w):
    return lax.dot_general(x.astype(jnp.bfloat16), w.astype(jnp.bfloat16),
                           (((1,), (1,)), ((), ())),
                           preferred_element_type=jnp.float32)


def _dot(x, w):
    return lax.dot_general(x, w, (((1,), (0,)), ((), ())),
                           preferred_element_type=jnp.float32)


def _dot00(x, w):
    return lax.dot_general(x, w, (((0,), (0,)), ((), ())),
                           preferred_element_type=jnp.float32)


def _mlp_chunk(pt_ref, seg_ref, rat_ref, g1_ref, g2_ref, a1_ref, a2_ref,
               a3_ref, a3b_ref, g2b_ref, a2b_ref,
               erp_sc, qjp_sc, s_sc, fjt_sc):
    pt = pt_ref[...]
    seg = seg_ref[...].reshape(1, TB)
    rat = rat_ref[...].reshape(1, TB)
    oh_s = (seg == lax.broadcasted_iota(jnp.int32, (B, TB), 0)
            ).astype(jnp.float32)
    oh_r = (rat == lax.broadcasted_iota(jnp.int32, (NR_PAD, TB), 0)
            ).astype(jnp.float32)

    g1 = g1_ref[...]
    a1 = a1_ref[...]

    ec = _dot00(oh_r, erp_sc[...])
    qc = _dot00(oh_s, qjp_sc[...])

    NS_ = 4
    HB = TB // NS_
    g2 = g2_ref[...]
    a2 = a2_ref[...]
    a3 = a3_ref[...]
    g2b = g2b_ref[...]
    a2b = a2b_ref[...]
    sl = [slice(k * HB, (k + 1) * HB) for k in range(NS_)]
    hs = [jnp.maximum(_bdot_t(pt[s], g1[:, :D]) + ec[s], 0.0) for s in sl]
    fs = [jnp.maximum(_bdot_t(h, g2) + g2b, 0.0) for h in hs]
    xs = [jnp.maximum(_bdot_t(f, a1[:, :D]) + qc[s], 0.0)
          for f, s in zip(fs, sl)]
    ys = [jnp.maximum(_bdot_t(x, a2) + a2b, 0.0) for x in xs]
    s_sc[...] = jnp.concatenate(
        [_bdot_t(a3, y) for y in ys], axis=1) + a3b_ref[0, 0]
    fjt_sc[...] = jnp.concatenate(fs, axis=0)


def _state_update(seg, s_sc, fjt_sc, m_sc, l_sc, zacc_sc):
    oh_s = (seg == lax.broadcasted_iota(jnp.int32, (B, TB), 0)
            ).astype(jnp.float32)
    s_tok = s_sc[...]
    smat = jnp.where(oh_s > 0.0, s_tok, NEG)
    m_old = m_sc[...]
    m_new = jnp.maximum(m_old, jnp.max(smat, axis=1, keepdims=True))
    mc = jnp.maximum(m_new, CLAMP)
    scale = jnp.exp(m_old - mc)
    m_tok = _dot00(mc, oh_s)
    e_tok = jnp.exp(s_tok - m_tok)
    et = oh_s * e_tok
    l_sum = lax.dot_general(oh_s, e_tok, (((1,), (1,)), ((), ())),
                            preferred_element_type=jnp.float32)
    l_sc[...] = l_sc[...] * scale + l_sum
    m_sc[...] = m_new

    eye = (lax.broadcasted_iota(jnp.int32, (B, B), 0)
           == lax.broadcasted_iota(jnp.int32, (B, B), 1))
    dscale = jnp.where(eye, _dot(scale, jnp.ones((1, B), jnp.float32)), 0.0)
    zacc_sc[...] = _dot(dscale, zacc_sc[...]) + _dot(et, fjt_sc[...])


def _proj_init(er_ref, qj_ref, g1_ref, a1_ref, g1b_ref, a1b_ref,
               erp_sc, qjp_sc):
    erp_sc[...] = (_dot_t(er_ref[...], g1_ref[...][:, D:])
                   + g1b_ref[...])
    qjp_sc[...] = (_dot_t(qj_ref[...], a1_ref[...][:, D:])
                   + a1b_ref[...])


def _tc_body_a(pt_ref, qj_ref, seg_ref, rat_ref, segp_ref, er_ref, g1_ref,
               g2_ref, a1_ref, a2_ref, a3_ref, g1b_ref, g2b_ref, a1b_ref,
               a2b_ref, a3b_ref, m_ref, l_ref, zacc_ref, qjp_ref,
               m_sc, l_sc, zacc_sc, erp_sc, qjp_sc, s_sc, fjt_sc):
    i = pl.program_id(0)

    @pl.when(i == 0)
    def _():
        m_sc[...] = jnp.full((B, 1), NEG, jnp.float32)
        l_sc[...] = jnp.zeros((B, 1), jnp.float32)
        zacc_sc[...] = jnp.zeros((B, D), jnp.float32)
        s_sc[...] = jnp.full((1, TB), NEG, jnp.float32)
        fjt_sc[...] = jnp.zeros((TB, D), jnp.float32)
        _proj_init(er_ref, qj_ref, g1_ref, a1_ref, g1b_ref, a1b_ref,
                   erp_sc, qjp_sc)

    _state_update(segp_ref[...].reshape(1, TB), s_sc, fjt_sc,
                  m_sc, l_sc, zacc_sc)

    _mlp_chunk(pt_ref, seg_ref, rat_ref, g1_ref, g2_ref, a1_ref, a2_ref,
               a3_ref, a3b_ref, g2b_ref, a2b_ref,
               erp_sc, qjp_sc, s_sc, fjt_sc)

    @pl.when(i == NA - 1)
    def _():
        _state_update(seg_ref[...].reshape(1, TB), s_sc, fjt_sc,
                      m_sc, l_sc, zacc_sc)
        m_ref[...] = m_sc[...]
        l_ref[...] = l_sc[...]
        zacc_ref[...] = zacc_sc[...]
        qjp_ref[...] = qjp_sc[...]


def _tc_body_b(pt_ref, seg_ref, rat_ref, segp_ref, er_ref, g1_ref, g2_ref,
               a1_ref, a2_ref, a3_ref, g1b_ref, g2b_ref, a1b_ref, a2b_ref,
               a3b_ref, m_in, l_in, zacc_in, qjp_in, z_ref,
               m_sc, l_sc, zacc_sc, erp_sc, qjp_sc, s_sc, fjt_sc):
    i = pl.program_id(0)

    @pl.when(i == 0)
    def _():
        m_sc[...] = m_in[...]
        l_sc[...] = l_in[...]
        zacc_sc[...] = zacc_in[...]
        qjp_sc[...] = qjp_in[...]
        s_sc[...] = jnp.full((1, TB), NEG, jnp.float32)
        fjt_sc[...] = jnp.zeros((TB, D), jnp.float32)
        erp_sc[...] = (_dot_t(er_ref[...], g1_ref[...][:, D:])
                       + g1b_ref[...])

    _state_update(segp_ref[...].reshape(1, TB), s_sc, fjt_sc,
                  m_sc, l_sc, zacc_sc)

    _mlp_chunk(pt_ref, seg_ref, rat_ref, g1_ref, g2_ref, a1_ref, a2_ref,
               a3_ref, a3b_ref, g2b_ref, a2b_ref,
               erp_sc, qjp_sc, s_sc, fjt_sc)

    @pl.when(i == NB_ - 1)
    def _():
        _state_update(seg_ref[...].reshape(1, TB), s_sc, fjt_sc,
                      m_sc, l_sc, zacc_sc)
        eye = (lax.broadcasted_iota(jnp.int32, (B, B), 0)
               == lax.broadcasted_iota(jnp.int32, (B, B), 1))
        recip = 1.0 / jnp.maximum(l_sc[...], 1e-30)
        drec = jnp.where(eye, _dot(recip, jnp.ones((1, B), jnp.float32)), 0.0)
        z_ref[...] = _dot(drec, zacc_sc[...])


_SCRATCH = [
    pltpu.VMEM((B, 1), jnp.float32),
    pltpu.VMEM((B, 1), jnp.float32),
    pltpu.VMEM((B, D), jnp.float32),
    pltpu.VMEM((NR_PAD, D), jnp.float32),
    pltpu.VMEM((B, D), jnp.float32),
    pltpu.VMEM((1, TB), jnp.float32),
    pltpu.VMEM((TB, D), jnp.float32),
]

_full = lambda s: pl.BlockSpec(s, lambda i: tuple(0 for _ in s))
_W_SPECS = [
    _full((NR_PAD, D)), _full((D, 2 * D)), _full((D, D)), _full((D, 2 * D)),
    _full((D, D)), _full((1, D)), _full((1, D)), _full((1, D)), _full((1, D)),
    _full((1, D)), _full((1, 1)),
]
_STATE_SHAPES = [
    jax.ShapeDtypeStruct((B, 1), jnp.float32),
    jax.ShapeDtypeStruct((B, 1), jnp.float32),
    jax.ShapeDtypeStruct((B, D), jnp.float32),
    jax.ShapeDtypeStruct((B, D), jnp.float32),
]
_STATE_SPECS = [_full((B, 1)), _full((B, 1)), _full((B, D)), _full((B, D))]


def _tc_pass_a(pt, qj, seg3, rat3, er_pad, *weights):
    return pl.pallas_call(
        _tc_body_a,
        grid=(NA,),
        in_specs=[
            pl.BlockSpec((TB, D), lambda i: (i, 0)),
            _full((B, D)),
            pl.BlockSpec((1, 1, TB), lambda i: (i, 0, 0)),
            pl.BlockSpec((1, 1, TB), lambda i: (i, 0, 0)),
            pl.BlockSpec((1, 1, TB), lambda i: (jnp.maximum(i - 1, 0), 0, 0)),
        ] + _W_SPECS,
        out_specs=_STATE_SPECS,
        out_shape=_STATE_SHAPES,
        scratch_shapes=_SCRATCH,
    )(pt, qj, seg3, rat3, seg3, er_pad, *weights)


def _tc_pass_b(pt, seg3, rat3, er_pad, state, *weights):
    return pl.pallas_call(
        _tc_body_b,
        grid=(NB_,),
        in_specs=[
            pl.BlockSpec((TB, D), lambda i: (i, 0)),
            pl.BlockSpec((1, 1, TB), lambda i: (i + NA, 0, 0)),
            pl.BlockSpec((1, 1, TB), lambda i: (i + NA, 0, 0)),
            pl.BlockSpec((1, 1, TB),
                         lambda i: (jnp.maximum(i - 1, 0) + NA, 0, 0)),
        ] + _W_SPECS + _STATE_SPECS,
        out_specs=_full((B, D)),
        out_shape=jax.ShapeDtypeStruct((B, D), jnp.float32),
        scratch_shapes=_SCRATCH,
    )(pt, seg3, rat3, seg3, er_pad, *weights, *state)


def kernel(nodes_v, flat_users, flat_ratings, segment_ids, embed_u_w,
           embed_i_w, embed_r_w, g1_w, g1_b, g2_w, g2_b, a1_w, a1_b,
           a2_w, a2_b, a3_w, a3_b):
    sc_a = _make_sc_gather(True, 0, TA)
    sc_b = _make_sc_gather(False, TA, TBK)
    pt_a, qj = sc_a(embed_u_w, flat_users, embed_i_w, nodes_v)
    pt_b, = sc_b(embed_u_w, flat_users)

    seg3 = segment_ids.reshape(NBT, 1, TB)
    rat3 = flat_ratings.reshape(NBT, 1, TB)
    er_pad = jnp.zeros((NR_PAD, D), jnp.float32).at[:5].set(embed_r_w)
    weights = (g1_w, g2_w, a1_w, a2_w, a3_w,
               g1_b.reshape(1, D), g2_b.reshape(1, D), a1_b.reshape(1, D),
               a2_b.reshape(1, D), a3_b.reshape(1, 1))

    state = _tc_pass_a(pt_a, qj, seg3, rat3, er_pad, *weights)
    return _tc_pass_b(pt_b, seg3, rat3, er_pad, state, *weights)
